# baseline (speedup 1.0000x reference)
import functools

import jax
import jax.numpy as jnp
from jax import lax
from jax.experimental import pallas as pl
from jax.experimental.pallas import tpu as pltpu
from jax.experimental.pallas import tpu_sc as plsc

_NC = 2
_NS = 16
_L = 16
_NW = _NC * _NS


def _grab_body(CHW, HW, W, n_units, ppw, n_chunks,
               xr_hbm, chans_hbm, c0_hbm, c1_hbm, out_hbm,
               chans_v, c0_v, c1_v, rowidx_v, rows_v, sem):
    wid = lax.axis_index("s") * _NC + lax.axis_index("c")
    base = wid * ppw

    pltpu.sync_copy(chans_hbm, chans_v)
    pltpu.sync_copy(c0_hbm, c0_v)
    pltpu.sync_copy(c1_hbm, c1_v)

    for i in range(ppw // _L):
        p0 = base + i * _L
        u0 = lax.rem(p0, n_units)
        b = lax.div(p0, n_units)
        f = (b * CHW
             + chans_v[pl.ds(u0, _L)] * HW
             + c0_v[pl.ds(u0, _L)] * W
             + c1_v[pl.ds(u0, _L)])
        rowidx_v[i // 8, pl.ds((i % 8) * _L, _L)] = f

    copies = [
        pltpu.async_copy(
            xr_hbm.at[rowidx_v.at[j]],
            rows_v.at[pl.ds(j * 128, 128)],
            sem,
        )
        for j in range(n_chunks)
    ]
    for cp in copies:
        cp.wait()

    pltpu.sync_copy(rows_v, out_hbm.at[pl.ds(base, ppw)])


def kernel(x, chans, coords):
    B, C, H, W = x.shape
    n_units = chans.shape[0]
    total = B * n_units
    assert total % (_NW * _L) == 0
    assert n_units % _L == 0
    ppw = total // _NW
    assert ppw % 128 == 0
    n_chunks = ppw // 128

    xr = x.reshape(-1)
    chans32 = chans.astype(jnp.int32)
    c0 = coords[:, 0].astype(jnp.int32)
    c1 = coords[:, 1].astype(jnp.int32)

    mesh = plsc.VectorSubcoreMesh(
        core_axis_name="c", subcore_axis_name="s",
        num_cores=_NC, num_subcores=_NS,
    )
    body = functools.partial(_grab_body, C * H * W, H * W, W, n_units,
                             ppw, n_chunks)
    out = pl.kernel(
        body,
        out_type=jax.ShapeDtypeStruct((total,), jnp.float32),
        mesh=mesh,
        scratch_types=[
            pltpu.VMEM((n_units,), jnp.int32),
            pltpu.VMEM((n_units,), jnp.int32),
            pltpu.VMEM((n_units,), jnp.int32),
            pltpu.VMEM((n_chunks, 128), jnp.int32),
            pltpu.VMEM((ppw,), jnp.float32),
            pltpu.SemaphoreType.DMA,
        ],
    )(xr, chans32, c0, c1)
    return out.reshape(B, n_units)


# speedup vs baseline: 125.2914x; 125.2914x over previous
import functools

import jax
import jax.numpy as jnp
from jax import lax
from jax.experimental import pallas as pl
from jax.experimental.pallas import tpu as pltpu
from jax.experimental.pallas import tpu_sc as plsc

_NC = 2
_NS = 16
_L = 16
_NW = _NC * _NS


def _grab_body(H, W, n_units, ppw, n_chunks,
               xt_hbm, chans_hbm, c0_hbm, c1_hbm, out_hbm,
               chans_v, c0_v, c1_v, rowidx_v, rowbuf_v, out_v, sem):
    wid = lax.axis_index("s") * _NC + lax.axis_index("c")
    base = wid * ppw

    pltpu.sync_copy(chans_hbm, chans_v)
    pltpu.sync_copy(c0_hbm, c0_v)
    pltpu.sync_copy(c1_hbm, c1_v)

    for i in range(ppw // _L):
        p0 = base + i * _L
        u0 = lax.rem(p0, n_units)
        b = lax.div(p0, n_units)
        row = (b * H + c0_v[pl.ds(u0, _L)]) * W + c1_v[pl.ds(u0, _L)]
        rowidx_v[i // 8, pl.ds((i % 8) * _L, _L)] = row

    copies = [
        pltpu.async_copy(
            xt_hbm.at[rowidx_v.at[j]],
            rowbuf_v.at[pl.ds(j * 128, 128)],
            sem,
        )
        for j in range(n_chunks)
    ]
    for cp in copies:
        cp.wait()

    for i in range(ppw // _L):
        u0 = lax.rem(base + i * _L, n_units)
        pos = i * _L + lax.iota(jnp.int32, _L)
        ch = chans_v[pl.ds(u0, _L)]
        out_v[pl.ds(i * _L, _L)] = plsc.load_gather(rowbuf_v, [pos, ch])

    pltpu.sync_copy(out_v, out_hbm.at[pl.ds(base, ppw)])


def kernel(x, chans, coords):
    B, C, H, W = x.shape
    n_units = chans.shape[0]
    total = B * n_units
    assert total % (_NW * _L) == 0
    assert n_units % _L == 0
    ppw = total // _NW
    assert ppw % 128 == 0
    n_chunks = ppw // 128

    xt = jnp.transpose(x, (0, 2, 3, 1)).reshape(-1, C)
    chans32 = chans.astype(jnp.int32)
    c0 = coords[:, 0].astype(jnp.int32)
    c1 = coords[:, 1].astype(jnp.int32)

    mesh = plsc.VectorSubcoreMesh(
        core_axis_name="c", subcore_axis_name="s",
        num_cores=_NC, num_subcores=_NS,
    )
    body = functools.partial(_grab_body, H, W, n_units, ppw, n_chunks)
    out = pl.kernel(
        body,
        out_type=jax.ShapeDtypeStruct((total,), jnp.float32),
        mesh=mesh,
        compiler_params=pltpu.CompilerParams(needs_layout_passes=False),
        scratch_types=[
            pltpu.VMEM((n_units,), jnp.int32),
            pltpu.VMEM((n_units,), jnp.int32),
            pltpu.VMEM((n_units,), jnp.int32),
            pltpu.VMEM((ppw // 128, 128), jnp.int32),
            pltpu.VMEM((ppw, 128), jnp.float32),
            pltpu.VMEM((ppw,), jnp.float32),
            pltpu.SemaphoreType.DMA,
        ],
    )(xt, chans32, c0, c1)
    return out.reshape(B, n_units)


# speedup vs baseline: 130.9167x; 1.0449x over previous
import functools

import jax
import jax.numpy as jnp
from jax import lax
from jax.experimental import pallas as pl
from jax.experimental.pallas import tpu as pltpu
from jax.experimental.pallas import tpu_sc as plsc

_NC = 2
_NS = 16
_L = 16
_NW = _NC * _NS


def _grab_body(H, W, n_units, ppw, n_chunks,
               xt_hbm, chans_hbm, c0_hbm, c1_hbm, out_hbm,
               chans_v, c0_v, c1_v, rowidx_v, rowbuf_v, out_v, sem):
    wid = lax.axis_index("s") * _NC + lax.axis_index("c")
    base = wid * ppw

    st = [pltpu.async_copy(chans_hbm, chans_v, sem),
          pltpu.async_copy(c0_hbm, c0_v, sem),
          pltpu.async_copy(c1_hbm, c1_v, sem)]
    for cp in st:
        cp.wait()

    copies = []
    for i in range(ppw // _L):
        p0 = base + i * _L
        u0 = lax.rem(p0, n_units)
        b = lax.div(p0, n_units)
        row = (b * H + c0_v[pl.ds(u0, _L)]) * W + c1_v[pl.ds(u0, _L)]
        copies.append(pltpu.async_copy(
            xt_hbm.at[row],
            rowbuf_v.at[pl.ds(i * _L, _L)],
            sem,
        ))
    for cp in copies:
        cp.wait()

    for i in range(ppw // _L):
        u0 = lax.rem(base + i * _L, n_units)
        pos = i * _L + lax.iota(jnp.int32, _L)
        ch = chans_v[pl.ds(u0, _L)]
        out_v[pl.ds(i * _L, _L)] = plsc.load_gather(rowbuf_v, [pos, ch])

    pltpu.sync_copy(out_v, out_hbm.at[pl.ds(base, ppw)])


def kernel(x, chans, coords):
    B, C, H, W = x.shape
    n_units = chans.shape[0]
    total = B * n_units
    assert total % (_NW * _L) == 0
    assert n_units % _L == 0
    ppw = total // _NW
    assert ppw % 128 == 0
    n_chunks = ppw // 128

    xt = jnp.transpose(x, (0, 2, 3, 1)).reshape(-1, C)
    chans32 = chans.astype(jnp.int32)
    c0 = coords[:, 0].astype(jnp.int32)
    c1 = coords[:, 1].astype(jnp.int32)

    mesh = plsc.VectorSubcoreMesh(
        core_axis_name="c", subcore_axis_name="s",
        num_cores=_NC, num_subcores=_NS,
    )
    body = functools.partial(_grab_body, H, W, n_units, ppw, n_chunks)
    out = pl.kernel(
        body,
        out_type=jax.ShapeDtypeStruct((total,), jnp.float32),
        mesh=mesh,
        compiler_params=pltpu.CompilerParams(needs_layout_passes=False),
        scratch_types=[
            pltpu.VMEM((n_units,), jnp.int32),
            pltpu.VMEM((n_units,), jnp.int32),
            pltpu.VMEM((n_units,), jnp.int32),
            pltpu.VMEM((ppw // 128, 128), jnp.int32),
            pltpu.VMEM((ppw, 128), jnp.float32),
            pltpu.VMEM((ppw,), jnp.float32),
            pltpu.SemaphoreType.DMA,
        ],
    )(xt, chans32, c0, c1)
    return out.reshape(B, n_units)


# speedup vs baseline: 131.8764x; 1.0073x over previous
import functools

import jax
import jax.numpy as jnp
from jax import lax
from jax.experimental import pallas as pl
from jax.experimental.pallas import tpu as pltpu
from jax.experimental.pallas import tpu_sc as plsc

_NC = 2
_NS = 16
_L = 16
_NW = _NC * _NS


def _grab_body(H, W, n_units, ppw, n_chunks,
               xt_hbm, chans_hbm, coordsT_hbm, out_hbm,
               chans_v, c0_v, c1_v, rowidx_v, rowbuf_v, out_v, sem, sem2):
    wid = lax.axis_index("s") * _NC + lax.axis_index("c")
    base = wid * ppw

    cp_ch = pltpu.async_copy(chans_hbm, chans_v, sem2)
    st = [pltpu.async_copy(coordsT_hbm.at[0], c0_v, sem),
          pltpu.async_copy(coordsT_hbm.at[1], c1_v, sem)]
    for cp in st:
        cp.wait()

    copies = []
    for i in range(ppw // _L):
        p0 = base + i * _L
        u0 = lax.rem(p0, n_units)
        b = lax.div(p0, n_units)
        row = (b * H + c0_v[pl.ds(u0, _L)]) * W + c1_v[pl.ds(u0, _L)]
        copies.append(pltpu.async_copy(
            xt_hbm.at[row],
            rowbuf_v.at[pl.ds(i * _L, _L)],
            sem,
        ))
    cp_ch.wait()
    for cp in copies:
        cp.wait()

    for i in range(ppw // _L):
        u0 = lax.rem(base + i * _L, n_units)
        pos = i * _L + lax.iota(jnp.int32, _L)
        ch = chans_v[pl.ds(u0, _L)]
        out_v[pl.ds(i * _L, _L)] = plsc.load_gather(rowbuf_v, [pos, ch])

    pltpu.sync_copy(out_v, out_hbm.at[pl.ds(base, ppw)])


def kernel(x, chans, coords):
    B, C, H, W = x.shape
    n_units = chans.shape[0]
    total = B * n_units
    assert total % (_NW * _L) == 0
    assert n_units % _L == 0
    ppw = total // _NW
    assert ppw % 128 == 0
    n_chunks = ppw // 128

    xt = jnp.transpose(x, (0, 2, 3, 1)).reshape(-1, C)
    chans32 = chans.astype(jnp.int32)
    coordsT = jnp.transpose(coords.astype(jnp.int32))

    mesh = plsc.VectorSubcoreMesh(
        core_axis_name="c", subcore_axis_name="s",
        num_cores=_NC, num_subcores=_NS,
    )
    body = functools.partial(_grab_body, H, W, n_units, ppw, n_chunks)
    out = pl.kernel(
        body,
        out_type=jax.ShapeDtypeStruct((total,), jnp.float32),
        mesh=mesh,
        compiler_params=pltpu.CompilerParams(needs_layout_passes=False),
        scratch_types=[
            pltpu.VMEM((n_units,), jnp.int32),
            pltpu.VMEM((n_units,), jnp.int32),
            pltpu.VMEM((n_units,), jnp.int32),
            pltpu.VMEM((ppw // 128, 128), jnp.int32),
            pltpu.VMEM((ppw, 128), jnp.float32),
            pltpu.VMEM((ppw,), jnp.float32),
            pltpu.SemaphoreType.DMA,
            pltpu.SemaphoreType.DMA,
        ],
    )(xt, chans32, coordsT)
    return out.reshape(B, n_units)
